# baseline (speedup 1.0000x reference)
import functools

import jax
import jax.numpy as jnp
from jax import lax
from jax.experimental import pallas as pl
from jax.experimental.pallas import tpu as pltpu
from jax.experimental.pallas import tpu_sc as plsc

NUM_EMB = 1000000
DIM = 64
ROWS = 16384
COLS = 50
NUM_CORES = 2
NUM_SUBCORES = 16
NW = NUM_CORES * NUM_SUBCORES
R_PER_W = ROWS // NW
NBUF = 8
NFLY = 6
LAG = NBUF - NFLY

_mesh = plsc.VectorSubcoreMesh(
    core_axis_name="c", subcore_axis_name="s",
    num_cores=NUM_CORES, num_subcores=NUM_SUBCORES)


@functools.partial(
    pl.kernel,
    mesh=_mesh,
    out_type=jax.ShapeDtypeStruct((ROWS, COLS, DIM), jnp.float32),
    scratch_types=[
        pltpu.VMEM((R_PER_W, COLS), jnp.int32),
        pltpu.VMEM((NBUF, COLS, DIM), jnp.float32),
    ] + [pltpu.SemaphoreType.DMA] * (2 * NBUF),
    compiler_params=pltpu.CompilerParams(use_tc_tiling_on_sc=False),
)
def _embed_sc(idx_hbm, table_hbm, out_hbm, idx_v, rows_v, *sems):
    gsem = sems[:NBUF]
    osem = sems[NBUF:]
    wid = lax.axis_index("s") * NUM_CORES + lax.axis_index("c")
    row0 = wid * R_PER_W

    pltpu.sync_copy(idx_hbm.at[pl.ds(row0, R_PER_W)], idx_v)

    def fire_gather(r, b):
        pltpu.async_copy(table_hbm.at[idx_v.at[r]], rows_v.at[b], gsem[b])

    def wait_gather(r, b):
        pltpu.make_async_copy(table_hbm.at[idx_v.at[r]], rows_v.at[b],
                              gsem[b]).wait()

    def fire_out(r, b):
        pltpu.async_copy(rows_v.at[b], out_hbm.at[row0 + r], osem[b])

    def wait_out(r, b):
        pltpu.make_async_copy(rows_v.at[b], out_hbm.at[row0 + r],
                              osem[b]).wait()

    for b in range(NFLY):
        fire_gather(b, b)

    def body(q, carry):
        for b in range(NBUF):
            r = q * NBUF + b
            wait_gather(r, b)
            fire_out(r, b)

            @pl.when(r >= LAG)
            def _():
                wait_out(r - LAG, (b - LAG) % NBUF)

            @pl.when(r < R_PER_W - NFLY)
            def _():
                fire_gather(r + NFLY, (b + NFLY) % NBUF)
        return carry

    lax.fori_loop(0, R_PER_W // NBUF, body, 0)

    for i in range(LAG):
        r = R_PER_W - LAG + i
        wait_out(r, r % NBUF)


def kernel(input_, weight):
    return _embed_sc(input_.astype(jnp.int32), weight)
